# baseline (speedup 1.0000x reference)
import functools

import jax
import jax.numpy as jnp
from jax import lax
from jax.experimental import pallas as pl
from jax.experimental.pallas import tpu as pltpu
from jax.experimental.pallas import tpu_sc as plsc

_LOG2E2 = 2.0 * 1.4426950408889634


@functools.lru_cache(maxsize=None)
def _make_sc_gather(total, D):
    info = plsc.get_sparse_core_info()
    NC, NS = info.num_cores, info.num_subcores
    NW = NC * NS
    CH = 128
    assert total % (NW * CH) == 0
    n_g = total // (NW * CH)
    b_per_w = n_g * CH

    mesh = plsc.VectorSubcoreMesh(core_axis_name="c", subcore_axis_name="s")

    @functools.partial(
        pl.kernel,
        mesh=mesh,
        out_type=jax.ShapeDtypeStruct((total, D), jnp.float32),
        scratch_types=[
            pltpu.VMEM((n_g, CH), jnp.int32),
            pltpu.VMEM((CH, D), jnp.float32),
            pltpu.VMEM((CH, D), jnp.float32),
            pltpu.SemaphoreType.DMA,
            pltpu.SemaphoreType.DMA,
        ],
    )
    def gather_k(table_hbm, idx_hbm, out_hbm, idx_v, rows0, rows1, sem0, sem1):
        wid = lax.axis_index("s") * NC + lax.axis_index("c")
        base = wid * b_per_w
        pltpu.sync_copy(idx_hbm.at[wid], idx_v)
        bufs = (rows0, rows1)
        sems = (sem0, sem1)
        cps = [None, None]
        cps[0] = pltpu.async_copy(table_hbm.at[idx_v.at[0]], bufs[0], sems[0])
        for g in range(n_g):
            if g + 1 < n_g:
                cps[(g + 1) % 2] = pltpu.async_copy(
                    table_hbm.at[idx_v.at[g + 1]], bufs[(g + 1) % 2],
                    sems[(g + 1) % 2])
            cps[g % 2].wait()
            pltpu.sync_copy(bufs[g % 2], out_hbm.at[pl.ds(base + g * CH, CH)])

    def run(table, idx):
        idx3 = idx.reshape(NW, n_g, CH)
        return gather_k(table, idx3)

    return run


def _nrm(x):
    return x * lax.rsqrt(jnp.maximum(jnp.sum(x * x, axis=1, keepdims=True),
                                     1e-24))


def _es(an_ref, n_ref):
    nn = _nrm(n_ref[...])
    sim2 = lax.dot_general(an_ref[...], nn.astype(jnp.bfloat16),
                           (((1,), (1,)), ((), ())),
                           preferred_element_type=jnp.float32)
    return jnp.sum(jnp.exp2(sim2.astype(jnp.bfloat16)).astype(jnp.float32),
                   axis=1, keepdims=True)


def _head_body(a_ref, n_ref, out_ref, an_ref):
    j = pl.program_id(0)

    @pl.when(j == 0)
    def _():
        an_ref[...] = (_nrm(a_ref[...]) * _LOG2E2).astype(jnp.bfloat16)

    es = _es(an_ref, n_ref)

    @pl.when(j == 0)
    def _():
        out_ref[...] = es

    @pl.when(j > 0)
    def _():
        out_ref[...] += es


def _tail_body(which, sig_ref, prev_ref, accin_ref, a_ref, p_ref, n_ref,
               out_ref, acc_ref, an_ref):
    j = pl.program_id(0)
    nj = pl.num_programs(0)

    @pl.when(j == 0)
    def _():
        an_ref[...] = (_nrm(a_ref[...]) * _LOG2E2).astype(jnp.bfloat16)

    es = _es(an_ref, n_ref)

    @pl.when(j == 0)
    def _():
        acc_ref[...] = accin_ref[...] + es

    @pl.when(j > 0)
    def _():
        acc_ref[...] += es

    @pl.when(j == nj - 1)
    def _():
        an = _nrm(a_ref[...])
        pn = _nrm(p_ref[...])
        pos = jnp.sum(an * pn, axis=1, keepdims=True) * 2.0
        lse = jnp.log(acc_ref[...] + jnp.exp(pos))
        part = jnp.mean(lse - pos)
        s = sig_ref[0, which]
        contrib = part / (2.0 * s * s) + jnp.log(s)
        out_ref[...] = prev_ref[...] + jnp.reshape(contrib, (1, 1))


_PARAMS = pltpu.CompilerParams(dimension_semantics=("arbitrary",))


@functools.lru_cache(maxsize=None)
def _make_head(B, D, nj, noff, CB=4096):
    return pl.pallas_call(
        _head_body,
        grid=(nj,),
        in_specs=[
            pl.BlockSpec((B, D), lambda j: (0, 0)),
            pl.BlockSpec((CB, D), lambda j, o=noff: (o + j, 0)),
        ],
        out_specs=pl.BlockSpec((B, 1), lambda j: (0, 0)),
        out_shape=jax.ShapeDtypeStruct((B, 1), jnp.float32),
        scratch_shapes=[pltpu.VMEM((B, D), jnp.bfloat16)],
        compiler_params=_PARAMS,
    )


@functools.lru_cache(maxsize=None)
def _make_tail(which, B, D, nj, noff, CB=4096):
    return pl.pallas_call(
        functools.partial(_tail_body, which),
        grid=(nj,),
        in_specs=[
            pl.BlockSpec((1, 2), lambda j: (0, 0)),
            pl.BlockSpec((1, 1), lambda j: (0, 0)),
            pl.BlockSpec((B, 1), lambda j: (0, 0)),
            pl.BlockSpec((B, D), lambda j: (0, 0)),
            pl.BlockSpec((B, D), lambda j: (1, 0)),
            pl.BlockSpec((CB, D), lambda j, o=noff: (o + j, 0)),
        ],
        out_specs=pl.BlockSpec((1, 1), lambda j: (0, 0)),
        out_shape=jax.ShapeDtypeStruct((1, 1), jnp.float32),
        scratch_shapes=[
            pltpu.VMEM((B, 1), jnp.float32),
            pltpu.VMEM((B, D), jnp.bfloat16),
        ],
        compiler_params=_PARAMS,
    )


def kernel(embed, sigma, anchor_class, pos_class, neg_class,
           anchor_ingred, pos_ingred, neg_ingred):
    D = embed.shape[1]
    B = anchor_class.shape[0]
    K = neg_class.shape[0] // B
    CB = 4096
    BK = B * K
    half = (2 * B + BK) // 2
    nh = (half - 2 * B) // CB
    nt = (BK - nh * CB) // CB

    idx_c = jnp.concatenate([anchor_class, pos_class, neg_class])
    idx_i = jnp.concatenate([anchor_ingred, pos_ingred, neg_ingred])
    g_half = _make_sc_gather(half, D)
    g_full = _make_sc_gather(2 * B + BK, D)
    rows_c1 = g_half(embed, idx_c[:half])
    rows_c2 = g_half(embed, idx_c[half:])
    rows_i = g_full(embed, idx_i)

    sig2d = sigma.reshape(1, 2)
    zero = jnp.zeros((1, 1), jnp.float32)
    zacc = jnp.zeros((B, 1), jnp.float32)
    acc_c = _make_head(B, D, nh, 1, CB)(rows_c1, rows_c1)
    loss_c = _make_tail(0, B, D, nt, 0, CB)(
        sig2d, zero, acc_c, rows_c1, rows_c1, rows_c2)
    total = _make_tail(1, B, D, BK // CB, 1, CB)(
        sig2d, loss_c, zacc, rows_i, rows_i, rows_i)
    return total.reshape(())


# speedup vs baseline: 1.0126x; 1.0126x over previous
import functools

import jax
import jax.numpy as jnp
from jax import lax
from jax.experimental import pallas as pl
from jax.experimental.pallas import tpu as pltpu
from jax.experimental.pallas import tpu_sc as plsc

_LOG2E2 = 2.0 * 1.4426950408889634


@functools.lru_cache(maxsize=None)
def _make_sc_gather(total, D):
    info = plsc.get_sparse_core_info()
    NC, NS = info.num_cores, info.num_subcores
    NW = NC * NS
    CH = 128
    assert total % (NW * CH) == 0
    n_g = total // (NW * CH)
    b_per_w = n_g * CH

    mesh = plsc.VectorSubcoreMesh(core_axis_name="c", subcore_axis_name="s")

    @functools.partial(
        pl.kernel,
        mesh=mesh,
        out_type=jax.ShapeDtypeStruct((total, D), jnp.float32),
        scratch_types=[
            pltpu.VMEM((n_g, CH), jnp.int32),
            pltpu.VMEM((CH, D), jnp.float32),
            pltpu.VMEM((CH, D), jnp.float32),
            pltpu.SemaphoreType.DMA,
            pltpu.SemaphoreType.DMA,
        ],
    )
    def gather_k(table_hbm, idx_hbm, out_hbm, idx_v, rows0, rows1, sem0, sem1):
        wid = lax.axis_index("s") * NC + lax.axis_index("c")
        base = wid * b_per_w
        pltpu.sync_copy(idx_hbm.at[wid], idx_v)
        bufs = (rows0, rows1)
        sems = (sem0, sem1)
        cps = [None, None]
        cps[0] = pltpu.async_copy(table_hbm.at[idx_v.at[0]], bufs[0], sems[0])
        for g in range(n_g):
            if g + 1 < n_g:
                cps[(g + 1) % 2] = pltpu.async_copy(
                    table_hbm.at[idx_v.at[g + 1]], bufs[(g + 1) % 2],
                    sems[(g + 1) % 2])
            cps[g % 2].wait()
            pltpu.sync_copy(bufs[g % 2], out_hbm.at[pl.ds(base + g * CH, CH)])

    def run(table, idx):
        idx3 = idx.reshape(NW, n_g, CH)
        return gather_k(table, idx3)

    return run


def _nrm(x):
    return x * lax.rsqrt(jnp.maximum(jnp.sum(x * x, axis=1, keepdims=True),
                                     1e-24))


def _es(an_ref, n_ref):
    nn = _nrm(n_ref[...])
    sim2 = lax.dot_general(an_ref[...], nn.astype(jnp.bfloat16),
                           (((1,), (1,)), ((), ())),
                           preferred_element_type=jnp.float32)
    return jnp.sum(jnp.exp2(sim2.astype(jnp.bfloat16)).astype(jnp.float32),
                   axis=1, keepdims=True)


def _head_body(a_ref, n_ref, out_ref, an_ref):
    j = pl.program_id(0)

    @pl.when(j == 0)
    def _():
        an_ref[...] = (_nrm(a_ref[...]) * _LOG2E2).astype(jnp.bfloat16)

    es = _es(an_ref, n_ref)

    @pl.when(j == 0)
    def _():
        out_ref[...] = es

    @pl.when(j > 0)
    def _():
        out_ref[...] += es


def _tail_body(which, sig_ref, prev_ref, accin_ref, a_ref, p_ref, n_ref,
               out_ref, acc_ref, an_ref):
    j = pl.program_id(0)
    nj = pl.num_programs(0)

    @pl.when(j == 0)
    def _():
        an_ref[...] = (_nrm(a_ref[...]) * _LOG2E2).astype(jnp.bfloat16)

    es = _es(an_ref, n_ref)

    @pl.when(j == 0)
    def _():
        acc_ref[...] = accin_ref[...] + es

    @pl.when(j > 0)
    def _():
        acc_ref[...] += es

    @pl.when(j == nj - 1)
    def _():
        an = _nrm(a_ref[...])
        pn = _nrm(p_ref[...])
        pos = jnp.sum(an * pn, axis=1, keepdims=True) * 2.0
        lse = jnp.log(acc_ref[...] + jnp.exp(pos))
        part = jnp.mean(lse - pos)
        s = sig_ref[0, which]
        contrib = part / (2.0 * s * s) + jnp.log(s)
        out_ref[...] = prev_ref[...] + jnp.reshape(contrib, (1, 1))


_PARAMS = pltpu.CompilerParams(dimension_semantics=("arbitrary",))


@functools.lru_cache(maxsize=None)
def _make_head(B, D, nj, noff, CB=4096):
    return pl.pallas_call(
        _head_body,
        grid=(nj,),
        in_specs=[
            pl.BlockSpec((B, D), lambda j: (0, 0)),
            pl.BlockSpec((CB, D), lambda j, o=noff: (o + j, 0)),
        ],
        out_specs=pl.BlockSpec((B, 1), lambda j: (0, 0)),
        out_shape=jax.ShapeDtypeStruct((B, 1), jnp.float32),
        scratch_shapes=[pltpu.VMEM((B, D), jnp.bfloat16)],
        compiler_params=_PARAMS,
    )


@functools.lru_cache(maxsize=None)
def _make_tail(which, B, D, nj, noff, CB=4096):
    return pl.pallas_call(
        functools.partial(_tail_body, which),
        grid=(nj,),
        in_specs=[
            pl.BlockSpec((1, 2), lambda j: (0, 0)),
            pl.BlockSpec((1, 1), lambda j: (0, 0)),
            pl.BlockSpec((B, 1), lambda j: (0, 0)),
            pl.BlockSpec((B, D), lambda j: (0, 0)),
            pl.BlockSpec((B, D), lambda j: (1, 0)),
            pl.BlockSpec((CB, D), lambda j, o=noff: (o + j, 0)),
        ],
        out_specs=pl.BlockSpec((1, 1), lambda j: (0, 0)),
        out_shape=jax.ShapeDtypeStruct((1, 1), jnp.float32),
        scratch_shapes=[
            pltpu.VMEM((B, 1), jnp.float32),
            pltpu.VMEM((B, D), jnp.bfloat16),
        ],
        compiler_params=_PARAMS,
    )


def kernel(embed, sigma, anchor_class, pos_class, neg_class,
           anchor_ingred, pos_ingred, neg_ingred):
    D = embed.shape[1]
    B = anchor_class.shape[0]
    K = neg_class.shape[0] // B
    CB = 4096
    BK = B * K
    nj = BK // CB

    idx_c = jnp.concatenate([anchor_class, pos_class, neg_class])
    idx_i = jnp.concatenate([anchor_ingred, pos_ingred, neg_ingred])
    g_full = _make_sc_gather(2 * B + BK, D)
    rows_c = g_full(embed, idx_c)
    rows_i = g_full(embed, idx_i)

    sig2d = sigma.reshape(1, 2)
    zero = jnp.zeros((1, 1), jnp.float32)
    zacc = jnp.zeros((B, 1), jnp.float32)
    loss_c = _make_tail(0, B, D, nj, 1, CB)(
        sig2d, zero, zacc, rows_c, rows_c, rows_c)
    total = _make_tail(1, B, D, nj, 1, CB)(
        sig2d, loss_c, zacc, rows_i, rows_i, rows_i)
    return total.reshape(())


# speedup vs baseline: 1.0329x; 1.0200x over previous
import functools

import jax
import jax.numpy as jnp
from jax import lax
from jax.experimental import pallas as pl
from jax.experimental.pallas import tpu as pltpu
from jax.experimental.pallas import tpu_sc as plsc

_LOG2E2 = 2.0 * 1.4426950408889634


@functools.lru_cache(maxsize=None)
def _make_sc_gather(total, D):
    info = plsc.get_sparse_core_info()
    NC, NS = info.num_cores, info.num_subcores
    NW = NC * NS
    CH = 128
    assert total % (NW * CH) == 0
    n_g = total // (NW * CH)
    b_per_w = n_g * CH

    mesh = plsc.VectorSubcoreMesh(core_axis_name="c", subcore_axis_name="s")

    @functools.partial(
        pl.kernel,
        mesh=mesh,
        out_type=jax.ShapeDtypeStruct((total, D), jnp.float32),
        scratch_types=[
            pltpu.VMEM((n_g, CH), jnp.int32),
            pltpu.VMEM((CH, D), jnp.float32),
            pltpu.VMEM((CH, D), jnp.float32),
            pltpu.SemaphoreType.DMA,
            pltpu.SemaphoreType.DMA,
        ],
    )
    def gather_k(table_hbm, idx_hbm, out_hbm, idx_v, rows0, rows1, sem0, sem1):
        wid = lax.axis_index("s") * NC + lax.axis_index("c")
        base = wid * b_per_w
        pltpu.sync_copy(idx_hbm.at[wid], idx_v)
        bufs = (rows0, rows1)
        sems = (sem0, sem1)
        cps = [None, None]
        cps[0] = pltpu.async_copy(table_hbm.at[idx_v.at[0]], bufs[0], sems[0])
        for g in range(n_g):
            if g + 1 < n_g:
                cps[(g + 1) % 2] = pltpu.async_copy(
                    table_hbm.at[idx_v.at[g + 1]], bufs[(g + 1) % 2],
                    sems[(g + 1) % 2])
            cps[g % 2].wait()
            pltpu.sync_copy(bufs[g % 2], out_hbm.at[pl.ds(base + g * CH, CH)])

    def run(table, idx):
        idx3 = idx.reshape(NW, n_g, CH)
        return gather_k(table, idx3)

    return run


def _nrm(x):
    return x * lax.rsqrt(jnp.maximum(jnp.sum(x * x, axis=1, keepdims=True),
                                     1e-24))


def _es(an_ref, n_ref):
    nn = _nrm(n_ref[...])
    sim2 = lax.dot_general(an_ref[...], nn.astype(jnp.bfloat16),
                           (((1,), (1,)), ((), ())),
                           preferred_element_type=jnp.float32)
    return jnp.sum(jnp.exp2(sim2.astype(jnp.bfloat16)).astype(jnp.float32),
                   axis=1, keepdims=True)


def _tail_body(which, sig_ref, prev_ref, a_ref, p_ref, n_ref,
               out_ref, acc_ref, an_ref):
    j = pl.program_id(0)
    nj = pl.num_programs(0)

    @pl.when(j == 0)
    def _():
        an_ref[...] = (_nrm(a_ref[...]) * _LOG2E2).astype(jnp.bfloat16)

    es = _es(an_ref, n_ref)

    @pl.when(j == 0)
    def _():
        acc_ref[...] = es

    @pl.when(j > 0)
    def _():
        acc_ref[...] += es

    @pl.when(j == nj - 1)
    def _():
        an = _nrm(a_ref[...])
        pn = _nrm(p_ref[...])
        pos = jnp.sum(an * pn, axis=1, keepdims=True) * 2.0
        lse = jnp.log(acc_ref[...] + jnp.exp(pos))
        part = jnp.mean(lse - pos)
        s = sig_ref[0, which]
        contrib = part / (2.0 * s * s) + jnp.log(s)
        out_ref[...] = prev_ref[...] + jnp.reshape(contrib, (1, 1))


_PARAMS = pltpu.CompilerParams(dimension_semantics=("arbitrary",))


@functools.lru_cache(maxsize=None)
def _make_tail(which, B, D, nj, noff, CB=4096):
    return pl.pallas_call(
        functools.partial(_tail_body, which),
        grid=(nj,),
        in_specs=[
            pl.BlockSpec((1, 2), lambda j: (0, 0)),
            pl.BlockSpec((1, 1), lambda j: (0, 0)),
            pl.BlockSpec((B, D), lambda j: (0, 0)),
            pl.BlockSpec((B, D), lambda j: (1, 0)),
            pl.BlockSpec((CB, D), lambda j, o=noff: (o + j, 0)),
        ],
        out_specs=pl.BlockSpec((1, 1), lambda j: (0, 0)),
        out_shape=jax.ShapeDtypeStruct((1, 1), jnp.float32),
        scratch_shapes=[
            pltpu.VMEM((B, 1), jnp.float32),
            pltpu.VMEM((B, D), jnp.bfloat16),
        ],
        compiler_params=_PARAMS,
    )


def kernel(embed, sigma, anchor_class, pos_class, neg_class,
           anchor_ingred, pos_ingred, neg_ingred):
    D = embed.shape[1]
    B = anchor_class.shape[0]
    K = neg_class.shape[0] // B
    CB = 4096
    BK = B * K
    nj = BK // CB

    idx_c = jnp.concatenate([anchor_class, pos_class, neg_class])
    idx_i = jnp.concatenate([anchor_ingred, pos_ingred, neg_ingred])
    g_full = _make_sc_gather(2 * B + BK, D)
    rows_c = g_full(embed, idx_c)
    rows_i = g_full(embed, idx_i)

    sig2d = sigma.reshape(1, 2)
    zero = jnp.zeros((1, 1), jnp.float32)
    loss_c = _make_tail(0, B, D, nj, 1, CB)(
        sig2d, zero, rows_c, rows_c, rows_c)
    total = _make_tail(1, B, D, nj, 1, CB)(
        sig2d, loss_c, rows_i, rows_i, rows_i)
    return total.reshape(())
